# Initial kernel scaffold; baseline (speedup 1.0000x reference)
import jax
import jax.numpy as jnp
from jax.experimental import pallas as pl


def kernel(obj_x, obj_pos, agent_pos, obj_agent_edge_index, agent_edge_index, W1, b1, W2, b2, Wd, bd, Wm1, bm1, Wm2, bm2, Wfd, bfd):
    raise NotImplementedError("write your pallas kernel here")


# speedup vs baseline: 6.2601x; 6.2601x over previous
import jax
import jax.numpy as jnp
from jax import lax
from jax.experimental import pallas as pl
from jax.experimental.pallas import tpu as pltpu
from jax.experimental.pallas import tpu_sc as plsc

N_AGENTS = 10000
N_ROWS = 10240
EMB = 128
FEAT = 16
POS = 2
ORIG = 18
MAXOBJ = 16
HSLOT = 8
E_OA = 160000
E_AA = 160000
THRES = 0.02
NEG = -1e9

NC, NS = 2, 16
QW = HSLOT * EMB

K1 = 128
EPS1 = 10240
E_PAD1 = EPS1 * NS
HR = N_ROWS // NC
ACC1 = 5184
ZCH1 = 54
RPTZ1 = ACC1 // NS
RPTC1 = HR // NS
K2 = 32
EPS2 = 10048
E_PAD2 = EPS2 * NS
MROWS = 10112

ZCH = 8
RPT = MROWS // NS


def _prep_obj_body(x_ref, p_ref, w1f_ref, w1p_ref, b1_ref, u_ref):
    u_ref[...] = (jnp.dot(x_ref[...], w1f_ref[...],
                          preferred_element_type=jnp.float32)
                  + jnp.dot(p_ref[...], w1p_ref[...],
                            preferred_element_type=jnp.float32)
                  + b1_ref[...])


def _prep_agent_body(ap_ref, w1p_ref, wm1p_ref, v_ref, v2_ref):
    ap = ap_ref[...]
    v_ref[...] = jnp.dot(ap, w1p_ref[...], preferred_element_type=jnp.float32)
    v2_ref[...] = jnp.dot(ap, wm1p_ref[...], preferred_element_type=jnp.float32)


def _prep_w_body(wd3_ref, wm1_ref, bd2_ref, bm1_ref, wq_ref, bq_ref):
    wm1 = wm1_ref[...]
    for s in range(MAXOBJ):
        wq_ref[:, s, :] = jnp.dot(wd3_ref[:, s, :], wm1,
                                  preferred_element_type=jnp.float32)
    bq_ref[...] = jnp.dot(bd2_ref[...], wm1,
                          preferred_element_type=jnp.float32) + bm1_ref[...]


def _stage_b_body(part_ref, v2_ref, w2_ref, b2_ref, wdpx_ref, bdpx_ref,
                  wdpy_ref, bdpy_ref, wq_ref, bq_ref, q_ref):
    a = part_ref.shape[0]
    pooled = part_ref[...]
    enc = jnp.dot(pooled, w2_ref[...],
                  preferred_element_type=jnp.float32) + b2_ref[...]
    p0x = jnp.dot(enc, wdpx_ref[...],
                  preferred_element_type=jnp.float32) + bdpx_ref[...]
    p0y = jnp.dot(enc, wdpy_ref[...],
                  preferred_element_type=jnp.float32) + bdpy_ref[...]
    t2 = THRES * THRES
    v2t = v2_ref[...]
    for s in range(MAXOBJ):
        qs = (jnp.dot(enc, wq_ref[:, s, :],
                      preferred_element_type=jnp.float32)
              + bq_ref[s:s + 1, :] + v2t)
        if s > 0:
            dxj = p0x[:, :s] - p0x[:, s:s + 1]
            dyj = p0y[:, :s] - p0y[:, s:s + 1]
            near = jnp.maximum(jnp.sign(t2 - (dxj * dxj + dyj * dyj)), 0.0)
            cnt = jnp.sum(near, axis=1, keepdims=True)
            m = jnp.maximum(jnp.sign(0.5 - cnt), 0.0)
            qs = qs * m + (1.0 - m) * NEG
        q_ref[:, s // HSLOT, s % HSLOT, :] = qs


def _stage_d_body(part_ref, wm2_ref, bm2_ref, wfd_ref, bfd_ref,
                  dec_ref, batch_ref):
    a = part_ref.shape[1]
    pooled2 = part_ref[0] + part_ref[1]
    merged = jnp.dot(pooled2, wm2_ref[...],
                     preferred_element_type=jnp.float32) + bm2_ref[...]
    dec_ref[...] = jnp.dot(merged, wfd_ref[...],
                           preferred_element_type=jnp.float32) + bfd_ref[...]
    batch_ref[...] = (lax.broadcasted_iota(jnp.int32, (a, MAXOBJ), 0)
                      + pl.program_id(0) * a)


def _zero_acc(sid, zbuf, acc, width, chunk, rows_per_sub):
    def zrow(r, carry):
        for k in range(width // 16):
            zbuf[r, pl.ds(k * 16, 16)] = jnp.zeros((16,), jnp.float32)
        return carry

    lax.fori_loop(0, chunk, zrow, 0)

    def zcp(t, carry):
        pltpu.sync_copy(zbuf, acc.at[pl.ds(sid * rows_per_sub + t * chunk,
                                           chunk)])
        return carry

    lax.fori_loop(0, rows_per_sub // chunk, zcp, 0)


def _encode_sc_body(u_hbm, v_hbm, src_hbm, dst_hbm, out_hbm,
                    idxs, idxd, buf_u, buf_v, zbuf, acc,
                    sem_u, sem_v):
    cid = lax.axis_index("c")
    sid = lax.axis_index("s")
    lo = cid * HR

    _zero_acc(sid, zbuf, acc, EMB, ZCH1, RPTZ1)
    plsc.subcore_barrier()

    def chunk(t, carry):
        base = pl.multiple_of(sid * EPS1 + t * K1, K1)
        pltpu.sync_copy(src_hbm.at[pl.ds(base, K1)], idxs)
        pltpu.sync_copy(dst_hbm.at[pl.ds(base, K1)], idxd)
        cp1 = pltpu.async_copy(u_hbm.at[idxs], buf_u, sem_u)
        cp2 = pltpu.async_copy(v_hbm.at[idxd], buf_v, sem_v)
        cp1.wait()
        cp2.wait()

        def edge(r, c2):
            for k in range(EMB // 16):
                sl = pl.ds(k * 16, 16)
                buf_u[r, sl] = jnp.maximum(buf_u[r, sl] - buf_v[r, sl], 0.0)
            return c2

        lax.fori_loop(0, K1, edge, 0)
        for k in range(K1 // 16):
            sl = pl.ds(k * 16, 16)
            d = idxd[sl] - lo
            ok = (d >= 0) & (d < HR)
            idxd[sl] = jnp.where(ok, d, HR)
        pltpu.sync_copy(buf_u, acc.at[idxd], add=True)
        return carry

    lax.fori_loop(0, EPS1 // K1, chunk, 0)
    plsc.subcore_barrier()
    pltpu.sync_copy(acc.at[pl.ds(sid * RPTC1, RPTC1)],
                    out_hbm.at[pl.ds(cid * HR + sid * RPTC1, RPTC1)])


def _merge_sc_body(q_hbm, v2_hbm, src_hbm, dst_hbm, out_hbm,
                   idxs, idxd, idx2s, buf_q, buf_v, buf_h, zbuf, acc,
                   sem_q, sem_v):
    cid = lax.axis_index("c")
    sid = lax.axis_index("s")

    _zero_acc(sid, zbuf, acc, EMB, ZCH, RPT)
    plsc.subcore_barrier()

    def chunk(t, carry):
        base = pl.multiple_of(sid * EPS2 + t * K2, K2)
        pltpu.sync_copy(src_hbm.at[pl.ds(base, K2)], idxs)
        pltpu.sync_copy(dst_hbm.at[pl.ds(base, K2)], idxd)
        for k in range(K2 // 16):
            sl = pl.ds(k * 16, 16)
            idx2s[sl] = idxs[sl] * 2 + cid
        cp1 = pltpu.async_copy(q_hbm.at[idx2s], buf_q, sem_q)
        cp2 = pltpu.async_copy(v2_hbm.at[idxd], buf_v, sem_v)
        cp1.wait()
        cp2.wait()

        def edge(r, c2):
            vv = [buf_v[r, pl.ds(k * 16, 16)] for k in range(EMB // 16)]
            hh = [jnp.zeros((16,), jnp.float32) for _ in range(EMB // 16)]
            for s in range(HSLOT):
                for k in range(EMB // 16):
                    qv = buf_q[r, pl.ds(s * EMB + k * 16, 16)]
                    hh[k] = hh[k] + jnp.maximum(qv - vv[k], 0.0)
            for k in range(EMB // 16):
                buf_h[r, pl.ds(k * 16, 16)] = hh[k]
            return c2

        lax.fori_loop(0, K2, edge, 0)
        pltpu.sync_copy(buf_h, acc.at[idxd], add=True)
        return carry

    lax.fori_loop(0, EPS2 // K2, chunk, 0)
    plsc.subcore_barrier()
    pltpu.sync_copy(acc.at[pl.ds(sid * RPT, RPT)],
                    out_hbm.at[cid, pl.ds(sid * RPT, RPT)])


def _full(shape):
    return pl.BlockSpec(shape, lambda i: tuple(0 for _ in shape))


def kernel(obj_x, obj_pos, agent_pos, obj_agent_edge_index, agent_edge_index,
           W1, b1, W2, b2, Wd, bd, Wm1, bm1, Wm2, bm2, Wfd, bfd):
    n_obj = obj_x.shape[0]
    f32 = jnp.float32

    W1f, W1p = W1[:FEAT], W1[FEAT:]
    Wm1p = Wm1[FEAT:]
    Wd3 = Wd.reshape(EMB, MAXOBJ, ORIG)
    bd2 = bd.reshape(MAXOBJ, ORIG)
    Wdpx, Wdpy = Wd3[:, :, FEAT], Wd3[:, :, FEAT + 1]
    bdpx = bd2[:, FEAT].reshape(1, MAXOBJ)
    bdpy = bd2[:, FEAT + 1].reshape(1, MAXOBJ)
    b1r = b1.reshape(1, EMB)
    b2r = b2.reshape(1, EMB)
    bm1r = bm1.reshape(1, EMB)
    bm2r = bm2.reshape(1, EMB)
    bfdr = bfd.reshape(1, MAXOBJ * ORIG)
    ap_pad = jnp.pad(agent_pos, ((0, N_ROWS - N_AGENTS), (0, 0)))

    oa = obj_agent_edge_index.astype(jnp.int32)
    aa = agent_edge_index.astype(jnp.int32)
    pad1 = E_PAD1 - E_OA
    src1 = jnp.concatenate([oa[1], jnp.zeros((pad1,), jnp.int32)])
    dst1 = jnp.concatenate([oa[0], jnp.full((pad1,), N_AGENTS, jnp.int32)])
    pad2 = E_PAD2 - E_AA
    src2 = jnp.concatenate([aa[0], jnp.zeros((pad2,), jnp.int32)])
    dst2 = jnp.concatenate([aa[1], jnp.full((pad2,), N_AGENTS, jnp.int32)])

    a_obj = 400
    u = pl.pallas_call(
        _prep_obj_body,
        grid=(n_obj // a_obj,),
        in_specs=[pl.BlockSpec((a_obj, FEAT), lambda i: (i, 0)),
                  pl.BlockSpec((a_obj, POS), lambda i: (i, 0)),
                  _full((FEAT, EMB)), _full((POS, EMB)), _full((1, EMB))],
        out_specs=pl.BlockSpec((a_obj, EMB), lambda i: (i, 0)),
        out_shape=jax.ShapeDtypeStruct((n_obj, EMB), f32),
    )(obj_x, obj_pos, W1f, W1p, b1r)

    a_ag = 512
    v, v2 = pl.pallas_call(
        _prep_agent_body,
        grid=(N_ROWS // a_ag,),
        in_specs=[pl.BlockSpec((a_ag, POS), lambda i: (i, 0)),
                  _full((POS, EMB)), _full((POS, EMB))],
        out_specs=[pl.BlockSpec((a_ag, EMB), lambda i: (i, 0)),
                   pl.BlockSpec((a_ag, EMB), lambda i: (i, 0))],
        out_shape=[jax.ShapeDtypeStruct((N_ROWS, EMB), f32),
                   jax.ShapeDtypeStruct((N_ROWS, EMB), f32)],
    )(ap_pad, W1p, Wm1p)

    Wq, bq = pl.pallas_call(
        _prep_w_body,
        out_shape=[jax.ShapeDtypeStruct((EMB, MAXOBJ, EMB), f32),
                   jax.ShapeDtypeStruct((MAXOBJ, EMB), f32)],
    )(Wd3, Wm1, bd2, bm1r)

    mesh = plsc.VectorSubcoreMesh(core_axis_name="c", subcore_axis_name="s",
                                  num_cores=NC, num_subcores=NS)
    pooled = pl.kernel(
        _encode_sc_body,
        out_type=jax.ShapeDtypeStruct((N_ROWS, EMB), f32),
        mesh=mesh,
        scratch_types=[
            pltpu.VMEM((K1,), jnp.int32),
            pltpu.VMEM((K1,), jnp.int32),
            pltpu.VMEM((K1, EMB), f32),
            pltpu.VMEM((K1, EMB), f32),
            pltpu.VMEM((ZCH1, EMB), f32),
            pltpu.VMEM_SHARED((ACC1, EMB), f32),
            pltpu.SemaphoreType.DMA,
            pltpu.SemaphoreType.DMA,
        ],
    )(u, v, src1, dst1)

    a_b = 400
    q4 = pl.pallas_call(
        _stage_b_body,
        grid=(N_AGENTS // a_b,),
        in_specs=[pl.BlockSpec((a_b, EMB), lambda i: (i, 0)),
                  pl.BlockSpec((a_b, EMB), lambda i: (i, 0)),
                  _full((EMB, EMB)), _full((1, EMB)),
                  _full((EMB, MAXOBJ)), _full((1, MAXOBJ)),
                  _full((EMB, MAXOBJ)), _full((1, MAXOBJ)),
                  _full((EMB, MAXOBJ, EMB)), _full((MAXOBJ, EMB))],
        out_specs=pl.BlockSpec((a_b, NC, HSLOT, EMB), lambda i: (i, 0, 0, 0)),
        out_shape=jax.ShapeDtypeStruct((N_AGENTS, NC, HSLOT, EMB), f32),
    )(pooled, v2, W2, b2r, Wdpx, bdpx, Wdpy, bdpy, Wq, bq)
    q2 = q4.reshape(N_AGENTS * NC, QW)

    part2 = pl.kernel(
        _merge_sc_body,
        out_type=jax.ShapeDtypeStruct((NC, MROWS, EMB), f32),
        mesh=mesh,
        scratch_types=[
            pltpu.VMEM((K2,), jnp.int32),
            pltpu.VMEM((K2,), jnp.int32),
            pltpu.VMEM((K2,), jnp.int32),
            pltpu.VMEM((K2, QW), f32),
            pltpu.VMEM((K2, EMB), f32),
            pltpu.VMEM((K2, EMB), f32),
            pltpu.VMEM((ZCH, EMB), f32),
            pltpu.VMEM_SHARED((MROWS, EMB), f32),
            pltpu.SemaphoreType.DMA,
            pltpu.SemaphoreType.DMA,
        ],
    )(q2, v2, src2, dst2)
    part2 = part2[:, :N_AGENTS]

    a_d = 400
    dec, batch2 = pl.pallas_call(
        _stage_d_body,
        grid=(N_AGENTS // a_d,),
        in_specs=[pl.BlockSpec((NC, a_d, EMB), lambda i: (0, i, 0)),
                  _full((EMB, EMB)), _full((1, EMB)),
                  _full((EMB, MAXOBJ * ORIG)), _full((1, MAXOBJ * ORIG))],
        out_specs=[pl.BlockSpec((a_d, MAXOBJ * ORIG), lambda i: (i, 0)),
                   pl.BlockSpec((a_d, MAXOBJ), lambda i: (i, 0))],
        out_shape=[jax.ShapeDtypeStruct((N_AGENTS, MAXOBJ * ORIG), f32),
                   jax.ShapeDtypeStruct((N_AGENTS, MAXOBJ), jnp.int32)],
    )(part2, Wm2, bm2r, Wfd, bfdr)

    decoded = dec.reshape(N_AGENTS * MAXOBJ, ORIG)
    batch = batch2.reshape(N_AGENTS * MAXOBJ)
    return decoded, batch


# speedup vs baseline: 6.4309x; 1.0273x over previous
import jax
import jax.numpy as jnp
from jax import lax
from jax.experimental import pallas as pl
from jax.experimental.pallas import tpu as pltpu
from jax.experimental.pallas import tpu_sc as plsc

N_AGENTS = 10000
N_ROWS = 10240
EMB = 128
FEAT = 16
POS = 2
ORIG = 18
MAXOBJ = 16
HSLOT = 8
E_OA = 160000
E_AA = 160000
THRES = 0.02
NEG = -1e9

NC, NS = 2, 16
QW = HSLOT * EMB

K1 = 128
EPS1 = 10240
E_PAD1 = EPS1 * NS
HR = N_ROWS // NC
ACC1 = 5184
ZCH1 = 54
RPTZ1 = ACC1 // NS
RPTC1 = HR // NS
K2 = 32
EPS2 = 10048
E_PAD2 = EPS2 * NS
MROWS = 10112

ZCH = 8
RPT = MROWS // NS


def _prep_obj_body(x_ref, p_ref, w1f_ref, w1p_ref, b1_ref, u_ref):
    u_ref[...] = (jnp.dot(x_ref[...], w1f_ref[...],
                          preferred_element_type=jnp.float32)
                  + jnp.dot(p_ref[...], w1p_ref[...],
                            preferred_element_type=jnp.float32)
                  + b1_ref[...])


def _prep_agent_body(ap_ref, w1p_ref, wm1p_ref, v_ref, v2_ref):
    ap = ap_ref[...]
    v_ref[...] = jnp.dot(ap, w1p_ref[...], preferred_element_type=jnp.float32)
    v2_ref[...] = jnp.dot(ap, wm1p_ref[...], preferred_element_type=jnp.float32)


def _prep_w_body(wd3_ref, wm1_ref, bd2_ref, bm1_ref, wq_ref, bq_ref):
    wm1 = wm1_ref[...]
    for s in range(MAXOBJ):
        wq_ref[:, s, :] = jnp.dot(wd3_ref[:, s, :], wm1,
                                  preferred_element_type=jnp.float32)
    bq_ref[...] = jnp.dot(bd2_ref[...], wm1,
                          preferred_element_type=jnp.float32) + bm1_ref[...]


def _stage_b_body(part_ref, v2_ref, w2_ref, b2_ref, wdpx_ref, bdpx_ref,
                  wdpy_ref, bdpy_ref, wq_ref, bq_ref, q_ref):
    a = part_ref.shape[0]
    pooled = part_ref[...]
    enc = jnp.dot(pooled, w2_ref[...],
                  preferred_element_type=jnp.float32) + b2_ref[...]
    p0x = jnp.dot(enc, wdpx_ref[...],
                  preferred_element_type=jnp.float32) + bdpx_ref[...]
    p0y = jnp.dot(enc, wdpy_ref[...],
                  preferred_element_type=jnp.float32) + bdpy_ref[...]
    t2 = THRES * THRES
    v2t = v2_ref[...]
    for s in range(MAXOBJ):
        qs = (jnp.dot(enc, wq_ref[:, s, :],
                      preferred_element_type=jnp.float32)
              + bq_ref[s:s + 1, :] + v2t)
        if s > 0:
            dxj = p0x[:, :s] - p0x[:, s:s + 1]
            dyj = p0y[:, :s] - p0y[:, s:s + 1]
            near = jnp.maximum(jnp.sign(t2 - (dxj * dxj + dyj * dyj)), 0.0)
            cnt = jnp.sum(near, axis=1, keepdims=True)
            m = jnp.maximum(jnp.sign(0.5 - cnt), 0.0)
            qs = qs * m + (1.0 - m) * NEG
        q_ref[:, s // HSLOT, s % HSLOT, :] = qs


def _stage_d_body(part_ref, wm2_ref, bm2_ref, wfd_ref, bfd_ref,
                  dec_ref, batch_ref):
    a = part_ref.shape[1]
    pooled2 = part_ref[0] + part_ref[1]
    merged = jnp.dot(pooled2, wm2_ref[...],
                     preferred_element_type=jnp.float32) + bm2_ref[...]
    dec_ref[...] = jnp.dot(merged, wfd_ref[...],
                           preferred_element_type=jnp.float32) + bfd_ref[...]
    batch_ref[...] = (lax.broadcasted_iota(jnp.int32, (a, MAXOBJ), 0)
                      + pl.program_id(0) * a)


def _zero_acc(sid, zbuf, acc, width, chunk, rows_per_sub):
    def zrow(r, carry):
        for k in range(width // 16):
            zbuf[r, pl.ds(k * 16, 16)] = jnp.zeros((16,), jnp.float32)
        return carry

    lax.fori_loop(0, chunk, zrow, 0)

    def zcp(t, carry):
        pltpu.sync_copy(zbuf, acc.at[pl.ds(sid * rows_per_sub + t * chunk,
                                           chunk)])
        return carry

    lax.fori_loop(0, rows_per_sub // chunk, zcp, 0)


def _encode_sc_body(u_hbm, v_hbm, src_hbm, dst_hbm, out_hbm,
                    idxs, idxd, buf_u, buf_v, zbuf, acc,
                    sem_u, sem_v):
    cid = lax.axis_index("c")
    sid = lax.axis_index("s")
    lo = cid * HR

    _zero_acc(sid, zbuf, acc, EMB, ZCH1, RPTZ1)
    plsc.subcore_barrier()

    def chunk(t, carry):
        base = pl.multiple_of(sid * EPS1 + t * K1, K1)
        pltpu.sync_copy(src_hbm.at[pl.ds(base, K1)], idxs)
        pltpu.sync_copy(dst_hbm.at[pl.ds(base, K1)], idxd)
        cp1 = pltpu.async_copy(u_hbm.at[idxs], buf_u, sem_u)
        cp2 = pltpu.async_copy(v_hbm.at[idxd], buf_v, sem_v)
        cp1.wait()
        cp2.wait()

        def edge(r, c2):
            for k in range(EMB // 16):
                sl = pl.ds(k * 16, 16)
                buf_u[r, sl] = jnp.maximum(buf_u[r, sl] - buf_v[r, sl], 0.0)
            return c2

        lax.fori_loop(0, K1, edge, 0)
        for k in range(K1 // 16):
            sl = pl.ds(k * 16, 16)
            d = idxd[sl] - lo
            ok = (d >= 0) & (d < HR)
            idxd[sl] = jnp.where(ok, d, HR)
        pltpu.sync_copy(buf_u, acc.at[idxd], add=True)
        return carry

    lax.fori_loop(0, EPS1 // K1, chunk, 0)
    plsc.subcore_barrier()
    pltpu.sync_copy(acc.at[pl.ds(sid * RPTC1, RPTC1)],
                    out_hbm.at[pl.ds(cid * HR + sid * RPTC1, RPTC1)])


def _merge_sc_body(q_hbm, v2_hbm, src_hbm, dst_hbm, out_hbm,
                   idxs, idxd, idx2s, buf_q, buf_v, buf_h, zbuf, acc,
                   sem_q, sem_v):
    cid = lax.axis_index("c")
    sid = lax.axis_index("s")

    _zero_acc(sid, zbuf, acc, EMB, ZCH, RPT)
    plsc.subcore_barrier()

    def chunk(t, carry):
        base = pl.multiple_of(sid * EPS2 + t * K2, K2)
        pltpu.sync_copy(src_hbm.at[pl.ds(base, K2)], idxs)
        pltpu.sync_copy(dst_hbm.at[pl.ds(base, K2)], idxd)
        for k in range(K2 // 16):
            sl = pl.ds(k * 16, 16)
            idx2s[sl] = idxs[sl] * 2 + cid
        cp1 = pltpu.async_copy(q_hbm.at[idx2s], buf_q, sem_q)
        cp2 = pltpu.async_copy(v2_hbm.at[idxd], buf_v, sem_v)
        cp1.wait()
        cp2.wait()

        def edge(r, c2):
            vv = [buf_v[r, pl.ds(k * 16, 16)] for k in range(EMB // 16)]
            hh = [jnp.zeros((16,), jnp.float32) for _ in range(EMB // 16)]
            for s in range(HSLOT):
                for k in range(EMB // 16):
                    q = buf_q[r, pl.ds(s * EMB + k * 16, 16)]
                    hh[k] = hh[k] + jnp.maximum(q, vv[k])
            for k in range(EMB // 16):
                buf_h[r, pl.ds(k * 16, 16)] = hh[k] - float(HSLOT) * vv[k]
            return c2

        lax.fori_loop(0, K2, edge, 0)
        pltpu.sync_copy(buf_h, acc.at[idxd], add=True)
        return carry

    lax.fori_loop(0, EPS2 // K2, chunk, 0)
    plsc.subcore_barrier()
    pltpu.sync_copy(acc.at[pl.ds(sid * RPT, RPT)],
                    out_hbm.at[cid, pl.ds(sid * RPT, RPT)])


def _full(shape):
    return pl.BlockSpec(shape, lambda i: tuple(0 for _ in shape))


def kernel(obj_x, obj_pos, agent_pos, obj_agent_edge_index, agent_edge_index,
           W1, b1, W2, b2, Wd, bd, Wm1, bm1, Wm2, bm2, Wfd, bfd):
    n_obj = obj_x.shape[0]
    f32 = jnp.float32

    W1f, W1p = W1[:FEAT], W1[FEAT:]
    Wm1p = Wm1[FEAT:]
    Wd3 = Wd.reshape(EMB, MAXOBJ, ORIG)
    bd2 = bd.reshape(MAXOBJ, ORIG)
    Wdpx, Wdpy = Wd3[:, :, FEAT], Wd3[:, :, FEAT + 1]
    bdpx = bd2[:, FEAT].reshape(1, MAXOBJ)
    bdpy = bd2[:, FEAT + 1].reshape(1, MAXOBJ)
    b1r = b1.reshape(1, EMB)
    b2r = b2.reshape(1, EMB)
    bm1r = bm1.reshape(1, EMB)
    bm2r = bm2.reshape(1, EMB)
    bfdr = bfd.reshape(1, MAXOBJ * ORIG)
    ap_pad = jnp.pad(agent_pos, ((0, N_ROWS - N_AGENTS), (0, 0)))

    oa = obj_agent_edge_index.astype(jnp.int32)
    aa = agent_edge_index.astype(jnp.int32)
    pad1 = E_PAD1 - E_OA
    src1 = jnp.concatenate([oa[1], jnp.zeros((pad1,), jnp.int32)])
    dst1 = jnp.concatenate([oa[0], jnp.full((pad1,), N_AGENTS, jnp.int32)])
    pad2 = E_PAD2 - E_AA
    src2 = jnp.concatenate([aa[0], jnp.zeros((pad2,), jnp.int32)])
    dst2 = jnp.concatenate([aa[1], jnp.full((pad2,), N_AGENTS, jnp.int32)])

    a_obj = 400
    u = pl.pallas_call(
        _prep_obj_body,
        grid=(n_obj // a_obj,),
        in_specs=[pl.BlockSpec((a_obj, FEAT), lambda i: (i, 0)),
                  pl.BlockSpec((a_obj, POS), lambda i: (i, 0)),
                  _full((FEAT, EMB)), _full((POS, EMB)), _full((1, EMB))],
        out_specs=pl.BlockSpec((a_obj, EMB), lambda i: (i, 0)),
        out_shape=jax.ShapeDtypeStruct((n_obj, EMB), f32),
    )(obj_x, obj_pos, W1f, W1p, b1r)

    a_ag = 512
    v, v2 = pl.pallas_call(
        _prep_agent_body,
        grid=(N_ROWS // a_ag,),
        in_specs=[pl.BlockSpec((a_ag, POS), lambda i: (i, 0)),
                  _full((POS, EMB)), _full((POS, EMB))],
        out_specs=[pl.BlockSpec((a_ag, EMB), lambda i: (i, 0)),
                   pl.BlockSpec((a_ag, EMB), lambda i: (i, 0))],
        out_shape=[jax.ShapeDtypeStruct((N_ROWS, EMB), f32),
                   jax.ShapeDtypeStruct((N_ROWS, EMB), f32)],
    )(ap_pad, W1p, Wm1p)

    Wq, bq = pl.pallas_call(
        _prep_w_body,
        out_shape=[jax.ShapeDtypeStruct((EMB, MAXOBJ, EMB), f32),
                   jax.ShapeDtypeStruct((MAXOBJ, EMB), f32)],
    )(Wd3, Wm1, bd2, bm1r)

    mesh = plsc.VectorSubcoreMesh(core_axis_name="c", subcore_axis_name="s",
                                  num_cores=NC, num_subcores=NS)
    pooled = pl.kernel(
        _encode_sc_body,
        out_type=jax.ShapeDtypeStruct((N_ROWS, EMB), f32),
        mesh=mesh,
        scratch_types=[
            pltpu.VMEM((K1,), jnp.int32),
            pltpu.VMEM((K1,), jnp.int32),
            pltpu.VMEM((K1, EMB), f32),
            pltpu.VMEM((K1, EMB), f32),
            pltpu.VMEM((ZCH1, EMB), f32),
            pltpu.VMEM_SHARED((ACC1, EMB), f32),
            pltpu.SemaphoreType.DMA,
            pltpu.SemaphoreType.DMA,
        ],
    )(u, v, src1, dst1)

    a_b = 400
    q4 = pl.pallas_call(
        _stage_b_body,
        grid=(N_AGENTS // a_b,),
        in_specs=[pl.BlockSpec((a_b, EMB), lambda i: (i, 0)),
                  pl.BlockSpec((a_b, EMB), lambda i: (i, 0)),
                  _full((EMB, EMB)), _full((1, EMB)),
                  _full((EMB, MAXOBJ)), _full((1, MAXOBJ)),
                  _full((EMB, MAXOBJ)), _full((1, MAXOBJ)),
                  _full((EMB, MAXOBJ, EMB)), _full((MAXOBJ, EMB))],
        out_specs=pl.BlockSpec((a_b, NC, HSLOT, EMB), lambda i: (i, 0, 0, 0)),
        out_shape=jax.ShapeDtypeStruct((N_AGENTS, NC, HSLOT, EMB),
                                       jnp.float32),
    )(pooled, v2, W2, b2r, Wdpx, bdpx, Wdpy, bdpy, Wq, bq)
    q2 = q4.reshape(N_AGENTS * NC, QW)

    part2 = pl.kernel(
        _merge_sc_body,
        out_type=jax.ShapeDtypeStruct((NC, MROWS, EMB), f32),
        mesh=mesh,
        scratch_types=[
            pltpu.VMEM((K2,), jnp.int32),
            pltpu.VMEM((K2,), jnp.int32),
            pltpu.VMEM((K2,), jnp.int32),
            pltpu.VMEM((K2, QW), jnp.float32),
            pltpu.VMEM((K2, EMB), f32),
            pltpu.VMEM((K2, EMB), f32),
            pltpu.VMEM((ZCH, EMB), f32),
            pltpu.VMEM_SHARED((MROWS, EMB), f32),
            pltpu.SemaphoreType.DMA,
            pltpu.SemaphoreType.DMA,
        ],
    )(q2, v2, src2, dst2)
    part2 = part2[:, :N_AGENTS]

    a_d = 400
    dec, batch2 = pl.pallas_call(
        _stage_d_body,
        grid=(N_AGENTS // a_d,),
        in_specs=[pl.BlockSpec((NC, a_d, EMB), lambda i: (0, i, 0)),
                  _full((EMB, EMB)), _full((1, EMB)),
                  _full((EMB, MAXOBJ * ORIG)), _full((1, MAXOBJ * ORIG))],
        out_specs=[pl.BlockSpec((a_d, MAXOBJ * ORIG), lambda i: (i, 0)),
                   pl.BlockSpec((a_d, MAXOBJ), lambda i: (i, 0))],
        out_shape=[jax.ShapeDtypeStruct((N_AGENTS, MAXOBJ * ORIG), f32),
                   jax.ShapeDtypeStruct((N_AGENTS, MAXOBJ), jnp.int32)],
    )(part2, Wm2, bm2r, Wfd, bfdr)

    decoded = dec.reshape(N_AGENTS * MAXOBJ, ORIG)
    batch = batch2.reshape(N_AGENTS * MAXOBJ)
    return decoded, batch
